# baseline (speedup 1.0000x reference)
import functools

import jax
import jax.numpy as jnp
from jax import lax
from jax.experimental import pallas as pl
from jax.experimental.pallas import tpu as pltpu
from jax.experimental.pallas import tpu_sc as plsc

B = 1024
T = 8192
C = 5
TD = T // 2
NC = 2
NS = 16
NW = NC * NS
ROWS_PER_W = B // NW
VSTEPS = T // 16
DSTEPS = TD // 16

_mesh = plsc.VectorSubcoreMesh(core_axis_name="c", subcore_axis_name="s")

_f32 = jnp.float32
_scratch = (
    [pltpu.VMEM((T,), _f32) for _ in range(4)]
    + [pltpu.VMEM((T,), _f32), pltpu.VMEM((T,), _f32), pltpu.VMEM((TD,), _f32)]
    + [pltpu.VMEM((T,), _f32) for _ in range(4)]
    + [pltpu.VMEM((T,), _f32), pltpu.VMEM((T,), _f32), pltpu.VMEM((TD,), _f32)]
    + [pltpu.SemaphoreType.DMA] * 4
)


@functools.partial(
    pl.kernel,
    mesh=_mesh,
    out_type=[
        jax.ShapeDtypeStruct((B * T,), jnp.float32),
        jax.ShapeDtypeStruct((B * TD,), jnp.float32),
    ],
    scratch_types=_scratch,
    compiler_params=pltpu.CompilerParams(needs_layout_passes=False),
)
def _sc_select(in_hbm, os_hbm, od_hbm,
               v1a, v2a, v3a, v4a, os1a, os2a, oda,
               v1b, v2b, v3b, v4b, os1b, os2b, odb,
               sin_a, sin_b, sout_a, sout_b):
    cid = lax.axis_index("c")
    sid = lax.axis_index("s")
    wid = sid * NC + cid
    row0 = wid * ROWS_PER_W
    lanes = lax.iota(jnp.int32, 16)
    lanes2 = lanes * 2
    zeros = jnp.zeros((16,), jnp.float32)
    inv_t = jnp.float32(1.0 / T)

    sets = (
        (v1a, v2a, v3a, v4a, os1a, os2a, oda, sin_a, sout_a),
        (v1b, v2b, v3b, v4b, os1b, os2b, odb, sin_b, sout_b),
    )
    lanes_roll1 = (lanes + 1) & 15
    lanes_zero = jnp.zeros((16,), jnp.int32)
    is15 = lanes == 15
    dnums = lax.GatherDimensionNumbers(
        offset_dims=(), collapsed_slice_dims=(0,), start_index_map=(0,))

    def take16(x, perm):
        return lax.gather(x, perm[:, None], dnums, slice_sizes=(1,),
                          mode=lax.GatherScatterMode.PROMISE_IN_BOUNDS)

    def start_in(row, st):
        sem = st[7]
        for c in range(4):
            pltpu.async_copy(in_hbm.at[c + 1, row], st[c], sem)

    def wait_in(st):
        sem = st[7]
        for c in range(4):
            pltpu.make_async_copy(in_hbm.at[0, 0], st[c], sem).wait()

    def start_out(row, st, pick1):
        sem = st[8]

        @pl.when(pick1)
        def _():
            pltpu.async_copy(st[4], os_hbm.at[pl.ds(row * T, T)], sem)

        @pl.when(jnp.logical_not(pick1))
        def _():
            pltpu.async_copy(st[5], os_hbm.at[pl.ds(row * T, T)], sem)

        pltpu.async_copy(st[6], od_hbm.at[pl.ds(row * TD, TD)], sem)

    def wait_out(st):
        sem = st[8]
        pltpu.make_async_copy(st[4], os_hbm.at[pl.ds(0, T)], sem).wait()
        pltpu.make_async_copy(st[6], od_hbm.at[pl.ds(0, TD)], sem).wait()

    def allsum(x):
        for shift in (8, 4, 2, 1):
            x = x + take16(x, (lanes + shift) & 15)
        return x

    def var_vec(s, q):
        sv = allsum(s) * inv_t
        return allsum(q) * inv_t - sv * sv

    def plane_var(vb):
        @plsc.parallel_loop(0, VSTEPS // 2, unroll=4,
                            carry=(zeros, zeros, zeros, zeros))
        def acc(j, a):
            s, q, s2, q2 = a
            x = vb[pl.ds(j * 32, 16)]
            y = vb[pl.ds(j * 32 + 16, 16)]
            return (s + x, q + x * x, s2 + y, q2 + y * y)

        s, q, s2, q2 = acc
        return var_vec(s + s2, q + q2)

    def var_smooth(vb, osc):
        x0 = vb[pl.ds(0, 16)]

        @plsc.parallel_loop(1, VSTEPS // 2, unroll=4,
                            carry=(x0, x0 * x0, zeros, zeros, x0))
        def acc(j, a):
            s, q, s2, q2, xp = a
            x = vb[pl.ds(j * 32 - 16, 16)]
            y = vb[pl.ds(j * 32, 16)]
            sh = take16(xp, lanes_roll1)
            fix = jnp.where(is15, take16(x, lanes_zero), sh)
            osc[pl.ds(j * 32 - 32, 16)] = (xp + fix) * 0.5
            sh2 = take16(x, lanes_roll1)
            fix2 = jnp.where(is15, take16(y, lanes_zero), sh2)
            osc[pl.ds(j * 32 - 16, 16)] = (x + fix2) * 0.5
            return (s + x, q + x * x, s2 + y, q2 + y * y, y)

        s, q, s2, q2, xp = acc
        xl = vb[pl.ds(T - 16, 16)]
        s = s + xl
        q = q + xl * xl
        sh = take16(xp, lanes_roll1)
        fix = jnp.where(is15, take16(xl, lanes_zero), sh)
        osc[pl.ds(T - 32, 16)] = (xp + fix) * 0.5
        sh2 = take16(xl, lanes_roll1)
        fix2 = jnp.where(is15, 0.0, sh2)
        osc[pl.ds(T - 16, 16)] = (xl + fix2) * 0.5
        return var_vec(s + s2, q + q2)

    def down_from(vb, odv):
        @plsc.parallel_loop(0, DSTEPS // 2, unroll=4, carry=lanes2)
        def _dn(j, idx):
            odv[pl.ds(j * 32, 16)] = plsc.load_gather(vb, [idx])
            odv[pl.ds(j * 32 + 16, 16)] = plsc.load_gather(vb, [idx + 32])
            return idx + 64

        del _dn

    def compute(st):
        v1, v2, v3, v4 = st[0], st[1], st[2], st[3]
        odv = st[6]
        var1 = var_smooth(v1, st[4])
        var2 = var_smooth(v2, st[5])
        var3 = plane_var(v3)
        var4 = plane_var(v4)
        pick1v = jnp.where(var1 >= var2, 1, 0)
        pick3v = jnp.where(var3 >= var4, 1, 0)
        pick1 = jnp.sum(pick1v) > 0
        pick3 = jnp.sum(pick3v) > 0

        @pl.when(pick3)
        def _():
            down_from(v3, odv)

        @pl.when(jnp.logical_not(pick3))
        def _():
            down_from(v4, odv)

        return pick1

    start_in(row0, sets[0])
    start_in(row0 + 1, sets[1])

    def pair(rr, carry):
        for k in (0, 1):
            st = sets[k]
            row = row0 + rr * 2 + k
            wait_in(st)

            @pl.when(rr > 0)
            def _():
                wait_out(st)

            pick1 = compute(st)
            start_out(row, st, pick1)
            nxt = jnp.minimum(row + 2, jnp.int32(B - 1))
            start_in(nxt, st)
        return carry

    lax.fori_loop(0, ROWS_PER_W // 2, pair, 0)

    for st in sets:
        wait_in(st)
        wait_out(st)


ROWS_PER_TC_BLK = 8


def _tc_ident_body(in_ref, out_ref):
    out_ref[...] = in_ref[0].reshape(ROWS_PER_TC_BLK * T)


_tc_ident = pl.pallas_call(
    _tc_ident_body,
    out_shape=jax.ShapeDtypeStruct((B * T,), jnp.float32),
    grid=(B // ROWS_PER_TC_BLK,),
    in_specs=[pl.BlockSpec((1, ROWS_PER_TC_BLK, T), lambda i: (0, i, 0))],
    out_specs=pl.BlockSpec((ROWS_PER_TC_BLK * T,), lambda i: (i,)),
)


def kernel(inputs):
    planar = jnp.transpose(inputs, (2, 0, 1))
    oi = _tc_ident(planar)
    osm, od = _sc_select(planar)
    return (
        oi.reshape(B, T, 1),
        osm.reshape(B, T, 1),
        od.reshape(B, TD, 1),
    )


# speedup vs baseline: 1.0049x; 1.0049x over previous
import functools

import jax
import jax.numpy as jnp
from jax import lax
from jax.experimental import pallas as pl
from jax.experimental.pallas import tpu as pltpu
from jax.experimental.pallas import tpu_sc as plsc

B = 1024
T = 8192
C = 5
TD = T // 2
NC = 2
NS = 16
NW = NC * NS
ROWS_PER_W = B // NW
VSTEPS = T // 16
DSTEPS = TD // 16

_mesh = plsc.VectorSubcoreMesh(core_axis_name="c", subcore_axis_name="s")

_f32 = jnp.float32
_scratch = (
    [pltpu.VMEM((T,), _f32) for _ in range(4)]
    + [pltpu.VMEM((T,), _f32), pltpu.VMEM((T,), _f32), pltpu.VMEM((TD,), _f32)]
    + [pltpu.VMEM((T,), _f32) for _ in range(4)]
    + [pltpu.VMEM((T,), _f32), pltpu.VMEM((T,), _f32), pltpu.VMEM((TD,), _f32)]
    + [pltpu.SemaphoreType.DMA] * 4
)


@functools.partial(
    pl.kernel,
    mesh=_mesh,
    out_type=[
        jax.ShapeDtypeStruct((B * T,), jnp.float32),
        jax.ShapeDtypeStruct((B * TD,), jnp.float32),
    ],
    scratch_types=_scratch,
    compiler_params=pltpu.CompilerParams(needs_layout_passes=False),
)
def _sc_select(in_hbm, os_hbm, od_hbm,
               v1a, v2a, v3a, v4a, os1a, os2a, oda,
               v1b, v2b, v3b, v4b, os1b, os2b, odb,
               sin_a, sin_b, sout_a, sout_b):
    cid = lax.axis_index("c")
    sid = lax.axis_index("s")
    wid = sid * NC + cid
    row0 = wid * ROWS_PER_W
    lanes = lax.iota(jnp.int32, 16)
    lanes2 = lanes * 2
    zeros = jnp.zeros((16,), jnp.float32)
    inv_t = jnp.float32(1.0 / T)

    sets = (
        (v1a, v2a, v3a, v4a, os1a, os2a, oda, sin_a, sout_a),
        (v1b, v2b, v3b, v4b, os1b, os2b, odb, sin_b, sout_b),
    )
    lanes_roll1 = (lanes + 1) & 15
    lanes_zero = jnp.zeros((16,), jnp.int32)
    is15 = lanes == 15
    dnums = lax.GatherDimensionNumbers(
        offset_dims=(), collapsed_slice_dims=(0,), start_index_map=(0,))

    def take16(x, perm):
        return lax.gather(x, perm[:, None], dnums, slice_sizes=(1,),
                          mode=lax.GatherScatterMode.PROMISE_IN_BOUNDS)

    def start_in(row, st):
        sem = st[7]
        for c in range(4):
            pltpu.async_copy(in_hbm.at[c + 1, row], st[c], sem)

    def wait_in(st):
        sem = st[7]
        for c in range(4):
            pltpu.make_async_copy(in_hbm.at[0, 0], st[c], sem).wait()

    def start_out(row, st, pick1):
        sem = st[8]

        @pl.when(pick1)
        def _():
            pltpu.async_copy(st[4], os_hbm.at[pl.ds(row * T, T)], sem)

        @pl.when(jnp.logical_not(pick1))
        def _():
            pltpu.async_copy(st[5], os_hbm.at[pl.ds(row * T, T)], sem)

        pltpu.async_copy(st[6], od_hbm.at[pl.ds(row * TD, TD)], sem)

    def wait_out(st):
        sem = st[8]
        pltpu.make_async_copy(st[4], os_hbm.at[pl.ds(0, T)], sem).wait()
        pltpu.make_async_copy(st[6], od_hbm.at[pl.ds(0, TD)], sem).wait()

    def allsum(x):
        for shift in (8, 4, 2, 1):
            x = x + take16(x, (lanes + shift) & 15)
        return x

    def var_vec(s, q):
        sv = allsum(s) * inv_t
        return allsum(q) * inv_t - sv * sv

    def plane_var(vb):
        @plsc.parallel_loop(0, VSTEPS // 2, unroll=4,
                            carry=(zeros, zeros, zeros, zeros))
        def acc(j, a):
            s, q, s2, q2 = a
            x = vb[pl.ds(j * 32, 16)]
            y = vb[pl.ds(j * 32 + 16, 16)]
            return (s + x, q + x * x, s2 + y, q2 + y * y)

        s, q, s2, q2 = acc
        return var_vec(s + s2, q + q2)

    def var_smooth_var(vb, vd, osc):
        x0 = vb[pl.ds(0, 16)]
        w0 = vd[pl.ds(0, 16)]

        @plsc.parallel_loop(1, VSTEPS // 2, unroll=2,
                            carry=(x0, x0 * x0, zeros, zeros, x0,
                                   w0, w0 * w0, zeros, zeros))
        def acc(j, a):
            s, q, s2, q2, xp, t, u, t2, u2 = a
            x = vb[pl.ds(j * 32 - 16, 16)]
            y = vb[pl.ds(j * 32, 16)]
            sh = take16(xp, lanes_roll1)
            fix = jnp.where(is15, take16(x, lanes_zero), sh)
            osc[pl.ds(j * 32 - 32, 16)] = (xp + fix) * 0.5
            sh2 = take16(x, lanes_roll1)
            fix2 = jnp.where(is15, take16(y, lanes_zero), sh2)
            osc[pl.ds(j * 32 - 16, 16)] = (x + fix2) * 0.5
            w = vd[pl.ds(j * 32 - 16, 16)]
            z = vd[pl.ds(j * 32, 16)]
            return (s + x, q + x * x, s2 + y, q2 + y * y, y,
                    t + w, u + w * w, t2 + z, u2 + z * z)

        s, q, s2, q2, xp, t, u, t2, u2 = acc
        xl = vb[pl.ds(T - 16, 16)]
        wl = vd[pl.ds(T - 16, 16)]
        s = s + xl
        q = q + xl * xl
        t = t + wl
        u = u + wl * wl
        sh = take16(xp, lanes_roll1)
        fix = jnp.where(is15, take16(xl, lanes_zero), sh)
        osc[pl.ds(T - 32, 16)] = (xp + fix) * 0.5
        sh2 = take16(xl, lanes_roll1)
        fix2 = jnp.where(is15, 0.0, sh2)
        osc[pl.ds(T - 16, 16)] = (xl + fix2) * 0.5
        return (var_vec(s + s2, q + q2), var_vec(t + t2, u + u2))

    def down_from(vb, odv):
        @plsc.parallel_loop(0, DSTEPS // 2, unroll=4, carry=lanes2)
        def _dn(j, idx):
            odv[pl.ds(j * 32, 16)] = plsc.load_gather(vb, [idx])
            odv[pl.ds(j * 32 + 16, 16)] = plsc.load_gather(vb, [idx + 32])
            return idx + 64

        del _dn

    def compute(st):
        v1, v2, v3, v4 = st[0], st[1], st[2], st[3]
        odv = st[6]
        var1, var3 = var_smooth_var(v1, v3, st[4])
        var2, var4 = var_smooth_var(v2, v4, st[5])
        pick1v = jnp.where(var1 >= var2, 1, 0)
        pick3v = jnp.where(var3 >= var4, 1, 0)
        pick1 = jnp.sum(pick1v) > 0
        pick3 = jnp.sum(pick3v) > 0

        @pl.when(pick3)
        def _():
            down_from(v3, odv)

        @pl.when(jnp.logical_not(pick3))
        def _():
            down_from(v4, odv)

        return pick1

    start_in(row0, sets[0])
    start_in(row0 + 1, sets[1])

    def pair(rr, carry):
        for k in (0, 1):
            st = sets[k]
            row = row0 + rr * 2 + k
            wait_in(st)

            @pl.when(rr > 0)
            def _():
                wait_out(st)

            pick1 = compute(st)
            start_out(row, st, pick1)
            nxt = jnp.minimum(row + 2, jnp.int32(B - 1))
            start_in(nxt, st)
        return carry

    lax.fori_loop(0, ROWS_PER_W // 2, pair, 0)

    for st in sets:
        wait_in(st)
        wait_out(st)


ROWS_PER_TC_BLK = 8


def _tc_ident_body(in_ref, out_ref):
    out_ref[...] = in_ref[0].reshape(ROWS_PER_TC_BLK * T)


_tc_ident = pl.pallas_call(
    _tc_ident_body,
    out_shape=jax.ShapeDtypeStruct((B * T,), jnp.float32),
    grid=(B // ROWS_PER_TC_BLK,),
    in_specs=[pl.BlockSpec((1, ROWS_PER_TC_BLK, T), lambda i: (0, i, 0))],
    out_specs=pl.BlockSpec((ROWS_PER_TC_BLK * T,), lambda i: (i,)),
)


def kernel(inputs):
    planar = jnp.transpose(inputs, (2, 0, 1))
    oi = _tc_ident(planar)
    osm, od = _sc_select(planar)
    return (
        oi.reshape(B, T, 1),
        osm.reshape(B, T, 1),
        od.reshape(B, TD, 1),
    )
